# baseline (speedup 1.0000x reference)
import functools

import jax
import jax.numpy as jnp
from jax import lax
from jax.experimental import pallas as pl
from jax.experimental.pallas import tpu as pltpu
from jax.experimental.pallas import tpu_sc as plsc


_NC, _NS = 2, 16
_NW = _NC * _NS
_C = 16
_L = 16
_NX = 4
_NP = 2


def kernel(x, pe_table):
    B, S, F = x.shape
    x2 = x.reshape(B * S, F)
    seq_per_w = S // _NW
    n_chunks = seq_per_w // _C
    n_units = n_chunks * B
    KPF = F // _L
    KPF_BITS = KPF.bit_length() - 1
    mesh = plsc.VectorSubcoreMesh(core_axis_name="c", subcore_axis_name="s")

    scratch = (
        [pltpu.VMEM((_C, F), jnp.float32) for _ in range(_NP)]
        + [pltpu.VMEM((_C, F), jnp.float32) for _ in range(_NX)]
        + [pltpu.SemaphoreType.DMA for _ in range(_NP)]
        + [pltpu.SemaphoreType.DMA for _ in range(2 * _NX)]
    )

    @functools.partial(
        pl.kernel,
        mesh=mesh,
        out_type=jax.ShapeDtypeStruct((B * S, F), jnp.float32),
        scratch_types=scratch,
    )
    def sc_add(x_hbm, pe_hbm, out_hbm, *refs):
        pebufs = refs[:_NP]
        xbufs = refs[_NP:_NP + _NX]
        pe_sems = refs[_NP + _NX:2 * _NP + _NX]
        x_sems = refs[2 * _NP + _NX:2 * _NP + 2 * _NX]
        o_sems = refs[2 * _NP + 2 * _NX:]
        wid = lax.axis_index("s") * _NC + lax.axis_index("c")
        s0 = wid * seq_per_w

        def load_pe(c):
            return pltpu.async_copy(
                pe_hbm.at[pl.ds(s0 + c * _C, _C)], pebufs[c % _NP], pe_sems[c % _NP]
            )

        def load_x(u):
            c, b = divmod(u, B)
            return pltpu.async_copy(
                x_hbm.at[pl.ds(b * S + s0 + c * _C, _C)], xbufs[u % _NX], x_sems[u % _NX]
            )

        def store_out(u):
            c, b = divmod(u, B)
            return pltpu.async_copy(
                xbufs[u % _NX], out_hbm.at[pl.ds(b * S + s0 + c * _C, _C)], o_sems[u % _NX]
            )

        pe_loads = {0: load_pe(0)}
        x_loads = {0: load_x(0)}
        stores = {}
        for u in range(n_units):
            c, b = divmod(u, B)
            if u + 1 < n_units:
                if u + 1 - _NX >= 0:
                    stores.pop(u + 1 - _NX).wait()
                x_loads[u + 1] = load_x(u + 1)
                if (u + 1) % B == 0:
                    pe_loads[c + 1] = load_pe(c + 1)
            x_loads.pop(u).wait()
            if b == 0:
                pe_loads.pop(c).wait()

            xbuf, pebuf = xbufs[u % _NX], pebufs[c % _NP]

            @plsc.parallel_loop(0, _C * KPF, unroll=8)
            def _(i):
                r = i >> KPF_BITS
                col = (i & (KPF - 1)) * _L
                plsc.addupdate(xbuf.at[r, pl.ds(col, _L)], pebuf[r, pl.ds(col, _L)])

            stores[u] = store_out(u)
        for u in sorted(stores):
            stores[u].wait()

    out = sc_add(x2, pe_table)
    return out.reshape(B, S, F)
